# baseline (speedup 1.0000x reference)
import jax
import jax.numpy as jnp
from jax import lax
from jax.experimental import pallas as pl
from jax.experimental.pallas import tpu as pltpu
from jax.experimental.pallas import tpu_sc as plsc

DIM = 32
BATCH = 16384
GROW = 128
REL_ROW = 256
NC, NS = 2, 16
NW = NC * NS
B_PER_W = BATCH // NW
CHUNK = 128
NCHUNK = B_PER_W // CHUNK
SANITY_EPS = 1e-08


def _sc_gather_kernel(ep_hbm, eb_hbm, rc_hbm, h_hbm, t_hbm, r_hbm,
                      o_hp, o_tp, o_hb, o_tb, o_rc,
                      hidx_v, tidx_v, ridx_v, buf_a, buf_b, buf_r,
                      sem_a, sem_b, sem_r, sem_oa, sem_ob, sem_or):
    wid = lax.axis_index("s") * NC + lax.axis_index("c")
    base = wid * B_PER_W
    sl = pl.ds(base, B_PER_W)
    pltpu.sync_copy(h_hbm.at[sl], hidx_v)
    pltpu.sync_copy(t_hbm.at[sl], tidx_v)
    pltpu.sync_copy(r_hbm.at[sl], ridx_v)
    for ci in range(NCHUNK):
        osl = pl.ds(base + ci * CHUNK, CHUNK)
        isl = pl.ds(ci * CHUNK, CHUNK)
        g1 = pltpu.async_copy(ep_hbm.at[hidx_v.at[isl]], buf_a, sem_a)
        g2 = pltpu.async_copy(ep_hbm.at[tidx_v.at[isl]], buf_b, sem_b)
        gr = pltpu.async_copy(rc_hbm.at[ridx_v.at[isl]], buf_r, sem_r)
        g1.wait()
        o1 = pltpu.async_copy(buf_a, o_hp.at[osl], sem_oa)
        g2.wait()
        o2 = pltpu.async_copy(buf_b, o_tp.at[osl], sem_ob)
        o1.wait()
        g3 = pltpu.async_copy(eb_hbm.at[hidx_v.at[isl]], buf_a, sem_a)
        o2.wait()
        g4 = pltpu.async_copy(eb_hbm.at[tidx_v.at[isl]], buf_b, sem_b)
        gr.wait()
        orl = pltpu.async_copy(buf_r, o_rc.at[osl], sem_or)
        g3.wait()
        o3 = pltpu.async_copy(buf_a, o_hb.at[osl], sem_oa)
        g4.wait()
        o4 = pltpu.async_copy(buf_b, o_tb.at[osl], sem_ob)
        o3.wait()
        o4.wait()
        orl.wait()


def _sc_gather(ep4, eb4, rel_cat, h4, t4, rels):
    mesh = plsc.VectorSubcoreMesh(core_axis_name="c", subcore_axis_name="s")
    f32 = jnp.float32
    out_type = [
        jax.ShapeDtypeStruct((BATCH, GROW), f32),
        jax.ShapeDtypeStruct((BATCH, GROW), f32),
        jax.ShapeDtypeStruct((BATCH, GROW), f32),
        jax.ShapeDtypeStruct((BATCH, GROW), f32),
        jax.ShapeDtypeStruct((BATCH, REL_ROW), f32),
    ]
    scratch_types = [
        pltpu.VMEM((B_PER_W,), jnp.int32),
        pltpu.VMEM((B_PER_W,), jnp.int32),
        pltpu.VMEM((B_PER_W,), jnp.int32),
        pltpu.VMEM((CHUNK, GROW), f32),
        pltpu.VMEM((CHUNK, GROW), f32),
        pltpu.VMEM((CHUNK, REL_ROW), f32),
        pltpu.SemaphoreType.DMA,
        pltpu.SemaphoreType.DMA,
        pltpu.SemaphoreType.DMA,
        pltpu.SemaphoreType.DMA,
        pltpu.SemaphoreType.DMA,
        pltpu.SemaphoreType.DMA,
    ]
    kern = pl.kernel(_sc_gather_kernel, out_type=out_type, mesh=mesh,
                     scratch_types=scratch_types)
    return kern(ep4, eb4, rel_cat, h4, t4, rels)


def _sel4(g, rem):
    return jnp.where(
        rem == 0, g[:, 0:32],
        jnp.where(rem == 1, g[:, 32:64],
                  jnp.where(rem == 2, g[:, 64:96], g[:, 96:128])))


def _tc_math_kernel(h_ref, t_ref, hp_ref, tp_ref, hb_ref, tb_ref, rc_ref,
                    o_ref):
    hrem = (h_ref[...] % 4)[:, None]
    trem = (t_ref[...] % 4)[:, None]
    hp = _sel4(hp_ref[...], hrem)
    tp = _sel4(tp_ref[...], trem)
    hb = _sel4(hb_ref[...], hrem)
    tb = _sel4(tb_ref[...], trem)
    rc = rc_ref[...]

    hbn = hb / jnp.maximum(
        jnp.sqrt(jnp.sum(hb * hb, axis=1, keepdims=True)), 1e-12)
    tbn = tb / jnp.maximum(
        jnp.sqrt(jnp.sum(tb * tb, axis=1, keepdims=True)), 1e-12)
    bumped_h = hp + tbn
    bumped_t = tp + hbn

    rb_h = rc[:, 0:32]
    rb_t = rc[:, 32:64]
    rs_h = rc[:, 64:96]
    rs_t = rc[:, 96:128]
    smx = rc[:, 128:130]
    sm = jnp.where(smx > 0, smx, jnp.exp(smx) - 1.0) + 1.0

    def pnorm(x):
        lg = jnp.log(jnp.abs(x) + SANITY_EPS)
        return x / jnp.exp(jnp.mean(lg, axis=1, keepdims=True))

    rd_h = sm[:, 0:1] * pnorm(rs_h)
    rd_t = sm[:, 1:2] * pnorm(rs_t)

    def box_dist(pt, base, delta):
        w = jnp.abs(delta)
        low = base - 0.5 * w
        high = base + 0.5 * w
        center = 0.5 * (low + high)
        width = high - low
        wp1 = width + 1.0
        inside = jnp.logical_and(pt >= low, pt <= high)
        d_in = jnp.abs(pt - center) / wp1
        d_out = jnp.abs(pt - center) * wp1 - 0.5 * width * (wp1 - 1.0 / wp1)
        return jnp.where(inside, d_in, d_out)

    d_h = box_dist(bumped_h, rb_h, rd_h)
    d_t = box_dist(bumped_t, rb_t, rd_t)
    score = -(jnp.sqrt(jnp.sum(d_h * d_h, axis=1))
              + jnp.sqrt(jnp.sum(d_t * d_t, axis=1)))
    o_ref[...] = score


def _tc_math(heads, tails, hp, tp, hb, tb, rc, interpret=False):
    bw = 2048
    grid = (BATCH // bw,)
    idx_spec = pl.BlockSpec((bw,), lambda i: (i,))
    ent_spec = pl.BlockSpec((bw, GROW), lambda i: (i, 0))
    return pl.pallas_call(
        _tc_math_kernel,
        grid=grid,
        in_specs=[idx_spec, idx_spec, ent_spec, ent_spec, ent_spec, ent_spec,
                  pl.BlockSpec((bw, REL_ROW), lambda i: (i, 0))],
        out_specs=pl.BlockSpec((bw,), lambda i: (i,)),
        out_shape=jax.ShapeDtypeStruct((BATCH,), jnp.float32),
        interpret=interpret,
    )(heads, tails, hp, tp, hb, tb, rc)


def kernel(entity_points, entity_bumps, rel_bases, rel_shapes, scale_mult,
           heads, tails, rels):
    nrel = rel_bases.shape[0]
    nent = entity_points.shape[0]
    ep4 = entity_points.reshape(nent // 4, GROW)
    eb4 = entity_bumps.reshape(nent // 4, GROW)
    rel_cat = jnp.concatenate([
        rel_bases.reshape(nrel, 2 * DIM),
        rel_shapes.reshape(nrel, 2 * DIM),
        scale_mult.reshape(nrel, 2),
        jnp.zeros((nrel, REL_ROW - 2 * DIM * 2 - 2), jnp.float32),
    ], axis=1)
    h4 = heads // 4
    t4 = tails // 4
    hp, tp, hb, tb, rc = _sc_gather(ep4, eb4, rel_cat, h4, t4, rels)
    return _tc_math(heads, tails, hp, tp, hb, tb, rc)


# speedup vs baseline: 1.0176x; 1.0176x over previous
import jax
import jax.numpy as jnp
from jax import lax
from jax.experimental import pallas as pl
from jax.experimental.pallas import tpu as pltpu
from jax.experimental.pallas import tpu_sc as plsc

DIM = 32
BATCH = 16384
PACK = 4
ROW = PACK * DIM
NC, NS = 2, 16
NW = NC * NS
B_PER_W = BATCH // NW
CHUNK = 128
NCHUNK = B_PER_W // CHUNK
SANITY_EPS = 1e-08


def _sc_gather_kernel(ep_hbm, eb_hbm, rp_hbm, h4_hbm, t4_hbm, r_hbm,
                      o_hp, o_tp, o_hb, o_tb, o_rc,
                      ih, it, ir, buf0, buf1, gs0, gs1, ws0, ws1):
    wid = lax.axis_index("s") * NC + lax.axis_index("c")
    base = wid * B_PER_W
    pltpu.sync_copy(h4_hbm.at[wid], ih)
    pltpu.sync_copy(t4_hbm.at[wid], it)
    pltpu.sync_copy(r_hbm.at[wid], ir)

    bufs = (buf0, buf1)
    gsem = (gs0, gs1)
    wsem = (ws0, ws1)
    jobs = []
    for tbl, idx, out in ((ep_hbm, ih, o_hp), (ep_hbm, it, o_tp),
                          (eb_hbm, ih, o_hb), (eb_hbm, it, o_tb),
                          (rp_hbm, ir, o_rc)):
        for c in range(NCHUNK):
            jobs.append((tbl, idx, out, c))

    g = [None, None]
    w = [None, None]
    prev = None
    for k, (tbl, idx, out, c) in enumerate(jobs):
        p = k % 2
        if w[p] is not None:
            w[p].wait()
        g[p] = pltpu.async_copy(tbl.at[idx.at[c]], bufs[p], gsem[p])
        if prev is not None:
            pout, pc, pp = prev
            g[pp].wait()
            w[pp] = pltpu.async_copy(
                bufs[pp], pout.at[pl.ds(base + pc * CHUNK, CHUNK)], wsem[pp])
        prev = (out, c, p)
    pout, pc, pp = prev
    g[pp].wait()
    w[pp] = pltpu.async_copy(
        bufs[pp], pout.at[pl.ds(base + pc * CHUNK, CHUNK)], wsem[pp])
    w[0].wait()
    w[1].wait()


def _sc_gather(ep4, eb4, relp, h4, t4, rr):
    mesh = plsc.VectorSubcoreMesh(core_axis_name="c", subcore_axis_name="s")
    f32 = jnp.float32
    out_type = [
        jax.ShapeDtypeStruct((BATCH, ROW), f32),
        jax.ShapeDtypeStruct((BATCH, ROW), f32),
        jax.ShapeDtypeStruct((BATCH, ROW), f32),
        jax.ShapeDtypeStruct((BATCH, ROW), f32),
        jax.ShapeDtypeStruct((BATCH, ROW), f32),
    ]
    scratch_types = [
        pltpu.VMEM((NCHUNK, CHUNK), jnp.int32),
        pltpu.VMEM((NCHUNK, CHUNK), jnp.int32),
        pltpu.VMEM((NCHUNK, CHUNK), jnp.int32),
        pltpu.VMEM((CHUNK, ROW), f32),
        pltpu.VMEM((CHUNK, ROW), f32),
        pltpu.SemaphoreType.DMA,
        pltpu.SemaphoreType.DMA,
        pltpu.SemaphoreType.DMA,
        pltpu.SemaphoreType.DMA,
    ]
    kern = pl.kernel(_sc_gather_kernel, out_type=out_type, mesh=mesh,
                     scratch_types=scratch_types)
    return kern(ep4, eb4, relp, h4, t4, rr)


def _rel_pack_kernel(rb_ref, rs_ref, sm_ref, o_ref):
    rb = rb_ref[...]
    rs = rs_ref[...]
    sm = sm_ref[...]
    smv = jnp.where(sm > 0, sm, jnp.exp(sm) - 1.0) + 1.0

    def pnorm(x):
        lg = jnp.log(jnp.abs(x) + SANITY_EPS)
        return x / jnp.exp(jnp.mean(lg, axis=1, keepdims=True))

    rd_h = smv[:, 0:1] * pnorm(rs[:, 0:DIM])
    rd_t = smv[:, 1:2] * pnorm(rs[:, DIM:2 * DIM])
    o_ref[...] = jnp.concatenate([rb, rd_h, rd_t], axis=1)


def _rel_pack(rb64, rs64, sm2, interpret=False):
    nrel = rb64.shape[0]
    return pl.pallas_call(
        _rel_pack_kernel,
        out_shape=jax.ShapeDtypeStruct((nrel, ROW), jnp.float32),
        interpret=interpret,
    )(rb64, rs64, sm2)


def _tc_math_kernel(h_ref, t_ref, hp_ref, tp_ref, hb_ref, tb_ref, rc_ref,
                    o_ref):
    bw = h_ref.shape[0]
    hm = jnp.reshape(lax.rem(h_ref[...], PACK), (bw, 1))
    tm = jnp.reshape(lax.rem(t_ref[...], PACK), (bw, 1))

    def sel(x, m):
        acc = jnp.where(m == 0, x[:, 0:DIM], 0.0)
        for k in range(1, PACK):
            acc = acc + jnp.where(m == k, x[:, k * DIM:(k + 1) * DIM], 0.0)
        return acc

    hp = sel(hp_ref[...], hm)
    tp = sel(tp_ref[...], tm)
    hb = sel(hb_ref[...], hm)
    tb = sel(tb_ref[...], tm)
    rc = rc_ref[...]

    hbn = hb / jnp.maximum(
        jnp.sqrt(jnp.sum(hb * hb, axis=1, keepdims=True)), 1e-12)
    tbn = tb / jnp.maximum(
        jnp.sqrt(jnp.sum(tb * tb, axis=1, keepdims=True)), 1e-12)
    bumped_h = hp + tbn
    bumped_t = tp + hbn

    rb_h = rc[:, 0:DIM]
    rb_t = rc[:, DIM:2 * DIM]
    rd_h = rc[:, 2 * DIM:3 * DIM]
    rd_t = rc[:, 3 * DIM:4 * DIM]

    def box_dist(pt, base, delta):
        w = jnp.abs(delta)
        low = base - 0.5 * w
        high = base + 0.5 * w
        center = 0.5 * (low + high)
        width = high - low
        wp1 = width + 1.0
        inside = jnp.logical_and(pt >= low, pt <= high)
        d_in = jnp.abs(pt - center) / wp1
        d_out = jnp.abs(pt - center) * wp1 - 0.5 * width * (wp1 - 1.0 / wp1)
        return jnp.where(inside, d_in, d_out)

    d_h = box_dist(bumped_h, rb_h, rd_h)
    d_t = box_dist(bumped_t, rb_t, rd_t)
    o_ref[...] = -(jnp.sqrt(jnp.sum(d_h * d_h, axis=1))
                   + jnp.sqrt(jnp.sum(d_t * d_t, axis=1)))


def _tc_math(heads, tails, hp, tp, hb, tb, rc, interpret=False):
    bw = 2048
    grid = (BATCH // bw,)
    row_spec = pl.BlockSpec((bw, ROW), lambda i: (i, 0))
    idx_spec = pl.BlockSpec((bw,), lambda i: (i,))
    return pl.pallas_call(
        _tc_math_kernel,
        grid=grid,
        in_specs=[idx_spec, idx_spec, row_spec, row_spec, row_spec, row_spec,
                  row_spec],
        out_specs=pl.BlockSpec((bw,), lambda i: (i,)),
        out_shape=jax.ShapeDtypeStruct((BATCH,), jnp.float32),
        interpret=interpret,
    )(heads, tails, hp, tp, hb, tb, rc)


def kernel(entity_points, entity_bumps, rel_bases, rel_shapes, scale_mult,
           heads, tails, rels):
    nrel = rel_bases.shape[0]
    ep4 = entity_points.reshape(-1, ROW)
    eb4 = entity_bumps.reshape(-1, ROW)
    relp = _rel_pack(rel_bases.reshape(nrel, 2 * DIM),
                     rel_shapes.reshape(nrel, 2 * DIM),
                     scale_mult.reshape(nrel, 2))
    h4 = (heads // PACK).reshape(NW, NCHUNK, CHUNK)
    t4 = (tails // PACK).reshape(NW, NCHUNK, CHUNK)
    rr = rels.reshape(NW, NCHUNK, CHUNK)
    hp, tp, hb, tb, rc = _sc_gather(ep4, eb4, relp, h4, t4, rr)
    return _tc_math(heads, tails, hp, tp, hb, tb, rc)


# speedup vs baseline: 1.0462x; 1.0281x over previous
import jax
import jax.numpy as jnp
from jax import lax
from jax.experimental import pallas as pl
from jax.experimental.pallas import tpu as pltpu
from jax.experimental.pallas import tpu_sc as plsc

DIM = 32
BATCH = 16384
ROW = 4 * DIM
NC, NS = 2, 16
NW = NC * NS
B_PER_W = BATCH // NW
CHUNK = 128
NCHUNK = B_PER_W // CHUNK
SANITY_EPS = 1e-08


def _sc_gather_kernel(ep_hbm, eb_hbm, rp_hbm, h_hbm, t_hbm, r_hbm,
                      o_hp, o_tp, o_hb, o_tb, o_rc,
                      ih, it, ir, ebuf0, ebuf1, rbuf0, rbuf1,
                      gs0, gs1, ws0, ws1):
    wid = lax.axis_index("s") * NC + lax.axis_index("c")
    base = wid * B_PER_W
    pltpu.sync_copy(h_hbm.at[wid], ih)
    pltpu.sync_copy(t_hbm.at[wid], it)
    pltpu.sync_copy(r_hbm.at[wid], ir)

    bufs = ((ebuf0, ebuf1), (rbuf0, rbuf1))
    gsem = (gs0, gs1)
    wsem = (ws0, ws1)
    jobs = []
    for bi, (tbl, idx, out) in enumerate(
            ((ep_hbm, ih, o_hp), (ep_hbm, it, o_tp),
             (eb_hbm, ih, o_hb), (eb_hbm, it, o_tb),
             (rp_hbm, ir, o_rc))):
        for c in range(NCHUNK):
            jobs.append((tbl, idx, out, c, 1 if bi == 4 else 0))

    g = [None, None]
    w = [None, None]
    prev = None
    for k, (tbl, idx, out, c, which) in enumerate(jobs):
        p = k % 2
        if w[p] is not None:
            w[p].wait()
        g[p] = pltpu.async_copy(tbl.at[idx.at[c]], bufs[which][p], gsem[p])
        if prev is not None:
            pout, pc, pp, pw = prev
            g[pp].wait()
            w[pp] = pltpu.async_copy(
                bufs[pw][pp], pout.at[pl.ds(base + pc * CHUNK, CHUNK)],
                wsem[pp])
        prev = (out, c, p, which)
    pout, pc, pp, pw = prev
    g[pp].wait()
    w[pp] = pltpu.async_copy(
        bufs[pw][pp], pout.at[pl.ds(base + pc * CHUNK, CHUNK)], wsem[pp])
    w[0].wait()
    w[1].wait()


def _sc_gather(ep, eb, relp, hh, tt, rr):
    mesh = plsc.VectorSubcoreMesh(core_axis_name="c", subcore_axis_name="s")
    f32 = jnp.float32
    out_type = [
        jax.ShapeDtypeStruct((BATCH, DIM), f32),
        jax.ShapeDtypeStruct((BATCH, DIM), f32),
        jax.ShapeDtypeStruct((BATCH, DIM), f32),
        jax.ShapeDtypeStruct((BATCH, DIM), f32),
        jax.ShapeDtypeStruct((BATCH, ROW), f32),
    ]
    scratch_types = [
        pltpu.VMEM((NCHUNK, CHUNK), jnp.int32),
        pltpu.VMEM((NCHUNK, CHUNK), jnp.int32),
        pltpu.VMEM((NCHUNK, CHUNK), jnp.int32),
        pltpu.VMEM((CHUNK, DIM), f32),
        pltpu.VMEM((CHUNK, DIM), f32),
        pltpu.VMEM((CHUNK, ROW), f32),
        pltpu.VMEM((CHUNK, ROW), f32),
        pltpu.SemaphoreType.DMA,
        pltpu.SemaphoreType.DMA,
        pltpu.SemaphoreType.DMA,
        pltpu.SemaphoreType.DMA,
    ]
    kern = pl.kernel(
        _sc_gather_kernel, out_type=out_type, mesh=mesh,
        scratch_types=scratch_types,
        compiler_params=pltpu.CompilerParams(use_tc_tiling_on_sc=False))
    return kern(ep, eb, relp, hh, tt, rr)


def _rel_pack_kernel(rb_ref, rs_ref, sm_ref, o_ref):
    rb = rb_ref[...]
    rs = rs_ref[...]
    sm = sm_ref[...]
    smv = jnp.where(sm > 0, sm, jnp.exp(sm) - 1.0) + 1.0

    def pnorm(x):
        lg = jnp.log(jnp.abs(x) + SANITY_EPS)
        return x / jnp.exp(jnp.mean(lg, axis=1, keepdims=True))

    rd_h = smv[:, 0:1] * pnorm(rs[:, 0:DIM])
    rd_t = smv[:, 1:2] * pnorm(rs[:, DIM:2 * DIM])
    o_ref[...] = jnp.concatenate([rb, rd_h, rd_t], axis=1)


def _rel_pack(rb64, rs64, sm2, interpret=False):
    nrel = rb64.shape[0]
    return pl.pallas_call(
        _rel_pack_kernel,
        out_shape=jax.ShapeDtypeStruct((nrel, ROW), jnp.float32),
        interpret=interpret,
    )(rb64, rs64, sm2)


def _tc_math_kernel(hp_ref, tp_ref, hb_ref, tb_ref, rc_ref, o_ref):
    hp = hp_ref[...]
    tp = tp_ref[...]
    hb = hb_ref[...]
    tb = tb_ref[...]
    rc = rc_ref[...]

    hbn = hb / jnp.maximum(
        jnp.sqrt(jnp.sum(hb * hb, axis=1, keepdims=True)), 1e-12)
    tbn = tb / jnp.maximum(
        jnp.sqrt(jnp.sum(tb * tb, axis=1, keepdims=True)), 1e-12)
    bumped_h = hp + tbn
    bumped_t = tp + hbn

    rb_h = rc[:, 0:DIM]
    rb_t = rc[:, DIM:2 * DIM]
    rd_h = rc[:, 2 * DIM:3 * DIM]
    rd_t = rc[:, 3 * DIM:4 * DIM]

    def box_dist(pt, base, delta):
        w = jnp.abs(delta)
        low = base - 0.5 * w
        high = base + 0.5 * w
        center = 0.5 * (low + high)
        width = high - low
        wp1 = width + 1.0
        inside = jnp.logical_and(pt >= low, pt <= high)
        d_in = jnp.abs(pt - center) / wp1
        d_out = jnp.abs(pt - center) * wp1 - 0.5 * width * (wp1 - 1.0 / wp1)
        return jnp.where(inside, d_in, d_out)

    d_h = box_dist(bumped_h, rb_h, rd_h)
    d_t = box_dist(bumped_t, rb_t, rd_t)
    o_ref[...] = -(jnp.sqrt(jnp.sum(d_h * d_h, axis=1))
                   + jnp.sqrt(jnp.sum(d_t * d_t, axis=1)))


def _tc_math(hp, tp, hb, tb, rc, interpret=False):
    bw = 2048
    grid = (BATCH // bw,)
    ent_spec = pl.BlockSpec((bw, DIM), lambda i: (i, 0))
    return pl.pallas_call(
        _tc_math_kernel,
        grid=grid,
        in_specs=[ent_spec, ent_spec, ent_spec, ent_spec,
                  pl.BlockSpec((bw, ROW), lambda i: (i, 0))],
        out_specs=pl.BlockSpec((bw,), lambda i: (i,)),
        out_shape=jax.ShapeDtypeStruct((BATCH,), jnp.float32),
        interpret=interpret,
    )(hp, tp, hb, tb, rc)


def kernel(entity_points, entity_bumps, rel_bases, rel_shapes, scale_mult,
           heads, tails, rels):
    nrel = rel_bases.shape[0]
    relp = _rel_pack(rel_bases.reshape(nrel, 2 * DIM),
                     rel_shapes.reshape(nrel, 2 * DIM),
                     scale_mult.reshape(nrel, 2))
    hh = heads.reshape(NW, NCHUNK, CHUNK)
    tt = tails.reshape(NW, NCHUNK, CHUNK)
    rr = rels.reshape(NW, NCHUNK, CHUNK)
    hp, tp, hb, tb, rc = _sc_gather(entity_points, entity_bumps, relp,
                                    hh, tt, rr)
    return _tc_math(hp, tp, hb, tb, rc)
